# baseline (speedup 1.0000x reference)
import functools

import jax
import jax.numpy as jnp
from jax import lax
from jax.experimental import pallas as pl
from jax.experimental.pallas import tpu as pltpu
from jax.experimental.pallas import tpu_sc as plsc

_M = 1024
_S = 32
_RADIUS2 = 0.2 ** 2
_EPS = 1e-5


def _fps(px8, py8, pz8, start_xyz):
    B = 4
    H = px8.shape[1]
    N = 2 * H

    def body(px_ref, py_ref, pz_ref, st_ref, out_ref,
             dist_scr, ax_scr, ay_scr, az_scr):
        px, py, pz = px_ref[...], py_ref[...], pz_ref[...]
        row = lax.broadcasted_iota(jnp.int32, (8, H), 0)
        io8 = lax.broadcasted_iota(jnp.int32, (8, H), 1) + \
            jnp.where(row >= B, H, 0)
        io_m = lax.broadcasted_iota(jnp.int32, (B, _M), 1)

        st = st_ref[...]

        def step(i, carry):
            cxb, cyb, czb = carry
            ax_scr[...] = jnp.where(io_m == i, cxb, ax_scr[...])
            ay_scr[...] = jnp.where(io_m == i, cyb, ay_scr[...])
            az_scr[...] = jnp.where(io_m == i, czb, az_scr[...])
            cx8 = jnp.concatenate([cxb, cxb], axis=0)
            cy8 = jnp.concatenate([cyb, cyb], axis=0)
            cz8 = jnp.concatenate([czb, czb], axis=0)
            d = (px - cx8) ** 2 + (py - cy8) ** 2 + (pz - cz8) ** 2
            dist = jnp.minimum(dist_scr[...], d)
            dist_scr[...] = dist
            m8 = jnp.max(dist, axis=1, keepdims=True)
            mb = jnp.maximum(m8[0:B], m8[B:])
            m8b = jnp.concatenate([mb, mb], axis=0)
            cand = jnp.where(dist == m8b, io8, N)
            n8 = jnp.min(cand, axis=1, keepdims=True)
            nb = jnp.minimum(n8[0:B], n8[B:])
            n8b = jnp.concatenate([nb, nb], axis=0)
            mask = io8 == n8b
            sx = jnp.sum(jnp.where(mask, px, 0.0), axis=1, keepdims=True)
            sy = jnp.sum(jnp.where(mask, py, 0.0), axis=1, keepdims=True)
            sz = jnp.sum(jnp.where(mask, pz, 0.0), axis=1, keepdims=True)
            cxb = sx[0:B] + sx[B:]
            cyb = sy[0:B] + sy[B:]
            czb = sz[0:B] + sz[B:]
            return cxb, cyb, czb

        dist_scr[...] = jnp.full((8, H), jnp.inf, jnp.float32)
        lax.fori_loop(
            0, _M, step, (st[:, 0:1], st[:, 1:2], st[:, 2:3]))
        out_ref[...] = jnp.concatenate(
            [ax_scr[...][:, None, :], ay_scr[...][:, None, :],
             az_scr[...][:, None, :]], axis=1)

    return pl.pallas_call(
        body,
        out_shape=jax.ShapeDtypeStruct((B, 3, _M), jnp.float32),
        scratch_shapes=[pltpu.VMEM((8, H), jnp.float32),
                        pltpu.VMEM((B, _M), jnp.float32),
                        pltpu.VMEM((B, _M), jnp.float32),
                        pltpu.VMEM((B, _M), jnp.float32)],
    )(px8, py8, pz8, start_xyz)


def _ballq(xyzT, new_xyz):
    B, _, N = xyzT.shape
    CB = 128
    NB = _M // CB

    def body(xyzT_ref, q_ref, out_ref, D_scr):
        b = pl.program_id(0)
        p = xyzT_ref[0]
        q = q_ref[0]
        dx = q[:, 0:1] - p[0:1, :]
        dy = q[:, 1:2] - p[1:2, :]
        dz = q[:, 2:3] - p[2:3, :]
        D = dx * dx + dy * dy + dz * dz
        iota = lax.broadcasted_iota(jnp.int32, (CB, N), 1)
        cnt = jnp.sum(jnp.where(D <= _RADIUS2, 1, 0).astype(jnp.int32),
                      axis=1, keepdims=True)
        l32 = lax.broadcasted_iota(jnp.int32, (CB, _S), 1)

        D_scr[...] = D

        def step(i, acc):
            Dm = D_scr[...]
            am = jnp.argmin(Dm, axis=1).astype(jnp.int32)[:, None]
            acc = acc + jnp.where(l32 == i, am, 0)
            D_scr[...] = jnp.where(iota == am, jnp.inf, Dm)
            return acc

        idxs = lax.fori_loop(
            0, _S, step, jnp.zeros((CB, _S), jnp.int32))
        first = idxs[:, 0:1]
        sel = jnp.where(l32 >= cnt, first, idxs)
        out_ref[0] = sel + b * N

    return pl.pallas_call(
        body,
        grid=(B, NB),
        in_specs=[
            pl.BlockSpec((1, 3, N), lambda b, i: (b, 0, 0)),
            pl.BlockSpec((1, CB, 3), lambda b, i: (b, i, 0)),
        ],
        out_specs=pl.BlockSpec((1, CB, _S), lambda b, i: (b, i, 0)),
        out_shape=jax.ShapeDtypeStruct((B, _M, _S), jnp.int32),
        scratch_shapes=[pltpu.VMEM((CB, N), jnp.float32)],
    )(xyzT, new_xyz)


def _sc_gather(table, idx):
    V, Dw = table.shape
    R = idx.shape[0]
    NW = 32
    per_w = R // NW
    CH = 512
    NCH = per_w // CH
    mesh = plsc.VectorSubcoreMesh(core_axis_name="c", subcore_axis_name="s")

    @functools.partial(
        pl.kernel,
        mesh=mesh,
        out_type=jax.ShapeDtypeStruct((R, Dw), jnp.float32),
        scratch_types=[
            pltpu.VMEM((CH,), jnp.int32),
            pltpu.VMEM((CH, Dw), jnp.float32),
            pltpu.SemaphoreType.DMA,
        ],
    )
    def gk(table_hbm, idx_hbm, out_hbm, idx_v, rows_v, sem):
        wid = lax.axis_index("s") * 2 + lax.axis_index("c")
        base = wid * per_w

        def chunk(c, carry):
            off = pl.multiple_of(base + c * CH, CH)
            pltpu.sync_copy(idx_hbm.at[pl.ds(off, CH)], idx_v)
            pltpu.async_copy(table_hbm.at[idx_v], rows_v, sem).wait()
            pltpu.sync_copy(rows_v, out_hbm.at[pl.ds(off, CH)])
            return carry

        lax.fori_loop(0, NCH, chunk, 0)

    return gk(table, idx)


def _mm1(x80, ctrp, w1p, w1a8, b1):
    R, K = x80.shape
    RB = 8192
    NB = R // RB
    G = RB // _S

    def body(x_ref, c_ref, w_ref, wa_ref, b_ref, y_ref, s_ref, q_ref):
        i = pl.program_id(0)
        y = jnp.dot(x_ref[...], w_ref[...],
                    preferred_element_type=jnp.float32)
        cy = jnp.dot(c_ref[...], wa_ref[...],
                     preferred_element_type=jnp.float32)
        y = (y.reshape(G, _S, 128) - cy[:, None, :]).reshape(RB, 128)
        y = y + b_ref[...]
        y_ref[...] = y
        ps = jnp.sum(y.reshape(RB // 8, 8, 128), axis=0)
        pq = jnp.sum((y * y).reshape(RB // 8, 8, 128), axis=0)

        @pl.when(i == 0)
        def _():
            s_ref[...] = ps
            q_ref[...] = pq

        @pl.when(i > 0)
        def _():
            s_ref[...] += ps
            q_ref[...] += pq

    return pl.pallas_call(
        body,
        grid=(NB,),
        in_specs=[
            pl.BlockSpec((RB, K), lambda i: (i, 0)),
            pl.BlockSpec((G, 8), lambda i: (i, 0)),
            pl.BlockSpec((K, 128), lambda i: (0, 0)),
            pl.BlockSpec((8, 128), lambda i: (0, 0)),
            pl.BlockSpec((1, 128), lambda i: (0, 0)),
        ],
        out_specs=[
            pl.BlockSpec((RB, 128), lambda i: (i, 0)),
            pl.BlockSpec((8, 128), lambda i: (0, 0)),
            pl.BlockSpec((8, 128), lambda i: (0, 0)),
        ],
        out_shape=[
            jax.ShapeDtypeStruct((R, 128), jnp.float32),
            jax.ShapeDtypeStruct((8, 128), jnp.float32),
            jax.ShapeDtypeStruct((8, 128), jnp.float32),
        ],
    )(x80, ctrp, w1p, w1a8, b1)


def _mmk(y_in, a, bc, wt, bias):
    R, K = y_in.shape
    Nc = wt.shape[1]
    RB = 8192
    NB = R // RB

    def body(x_ref, a_ref, c_ref, w_ref, b_ref, y_ref, s_ref, q_ref):
        i = pl.program_id(0)
        z = jnp.maximum(x_ref[...] * a_ref[...] + c_ref[...], 0.0)
        y = jnp.dot(z, w_ref[...], preferred_element_type=jnp.float32)
        y = y + b_ref[...]
        y_ref[...] = y
        ps = jnp.sum(y.reshape(RB // 8, 8, Nc), axis=0)
        pq = jnp.sum((y * y).reshape(RB // 8, 8, Nc), axis=0)

        @pl.when(i == 0)
        def _():
            s_ref[...] = ps
            q_ref[...] = pq

        @pl.when(i > 0)
        def _():
            s_ref[...] += ps
            q_ref[...] += pq

    return pl.pallas_call(
        body,
        grid=(NB,),
        in_specs=[
            pl.BlockSpec((RB, K), lambda i: (i, 0)),
            pl.BlockSpec((1, K), lambda i: (0, 0)),
            pl.BlockSpec((1, K), lambda i: (0, 0)),
            pl.BlockSpec((K, Nc), lambda i: (0, 0)),
            pl.BlockSpec((1, Nc), lambda i: (0, 0)),
        ],
        out_specs=[
            pl.BlockSpec((RB, Nc), lambda i: (i, 0)),
            pl.BlockSpec((8, Nc), lambda i: (0, 0)),
            pl.BlockSpec((8, Nc), lambda i: (0, 0)),
        ],
        out_shape=[
            jax.ShapeDtypeStruct((R, Nc), jnp.float32),
            jax.ShapeDtypeStruct((8, Nc), jnp.float32),
            jax.ShapeDtypeStruct((8, Nc), jnp.float32),
        ],
    )(y_in, a, bc, wt, bias)


def _final(y3, a, bc):
    R, Nc = y3.shape
    RB = 8192
    NB = R // RB
    G = RB // _S

    def body(x_ref, a_ref, c_ref, out_ref):
        z = jnp.maximum(x_ref[...] * a_ref[...] + c_ref[...], 0.0)
        out_ref[...] = jnp.max(z.reshape(G, _S, Nc), axis=1)

    return pl.pallas_call(
        body,
        grid=(NB,),
        in_specs=[
            pl.BlockSpec((RB, Nc), lambda i: (i, 0)),
            pl.BlockSpec((1, Nc), lambda i: (0, 0)),
            pl.BlockSpec((1, Nc), lambda i: (0, 0)),
        ],
        out_specs=pl.BlockSpec((G, Nc), lambda i: (i, 0)),
        out_shape=jax.ShapeDtypeStruct((R // _S, Nc), jnp.float32),
    )(y3, a, bc)


def kernel(xyz, features, W1, b1, g1, beta1, W2, b2, g2, beta2,
           W3, b3, g3, beta3):
    B, N, _ = xyz.shape
    C = features.shape[-1]

    xyzT = jnp.transpose(xyz, (0, 2, 1))
    idx0 = jax.random.randint(jax.random.key(7), (B,), 0, N)
    start_xyz = xyz[jnp.arange(B), idx0]

    H = N // 2
    planes8 = jnp.transpose(xyzT.reshape(B, 3, 2, H),
                            (1, 2, 0, 3)).reshape(3, 2 * B, H)
    oxyzT = _fps(planes8[0], planes8[1], planes8[2], start_xyz)
    new_xyz = jnp.transpose(oxyzT, (0, 2, 1))

    ball_idx = _ballq(xyzT, new_xyz)

    TW = 128
    table = jnp.concatenate(
        [xyz, jnp.zeros((B, N, 13), jnp.float32), features,
         jnp.zeros((B, N, TW - 16 - C), jnp.float32)],
        axis=-1).reshape(B * N, TW)
    gathered = _sc_gather(table, ball_idx.reshape(-1))

    ctrp = jnp.pad(new_xyz.reshape(B * _M, 3), ((0, 0), (0, 5)))
    w1p = jnp.zeros((TW, 128), jnp.float32)
    w1p = w1p.at[0:3].set(W1[:, 0:3].T).at[16:16 + C].set(W1[:, 3:3 + C].T)
    w1a8 = jnp.zeros((8, 128), jnp.float32).at[0:3].set(W1[:, 0:3].T)

    cnt = float(B * _M * _S)

    def fold(s, q, g, beta):
        mean = jnp.sum(s, axis=0) / cnt
        var = jnp.sum(q, axis=0) / cnt - mean * mean
        a = g / jnp.sqrt(var + _EPS)
        return (a.reshape(1, -1), (beta - mean * a).reshape(1, -1))

    y1, s1, q1 = _mm1(gathered, ctrp, w1p, w1a8, b1.reshape(1, -1))
    a1, c1 = fold(s1, q1, g1, beta1)
    y2, s2, q2 = _mmk(y1, a1, c1, W2.T, b2.reshape(1, -1))
    a2, c2 = fold(s2, q2, g2, beta2)
    y3, s3, q3 = _mmk(y2, a2, c2, W3.T, b3.reshape(1, -1))
    a3, c3 = fold(s3, q3, g3, beta3)
    feat_out = _final(y3, a3, c3).reshape(B, _M, W3.shape[0])

    return new_xyz, feat_out


# speedup vs baseline: 1.1551x; 1.1551x over previous
import functools

import jax
import jax.numpy as jnp
from jax import lax
from jax.experimental import pallas as pl
from jax.experimental.pallas import tpu as pltpu
from jax.experimental.pallas import tpu_sc as plsc

_M = 1024
_S = 32
_RADIUS2 = 0.2 ** 2
_EPS = 1e-5


def _fps(px8, py8, pz8, start_xyz):
    B = 4
    H = px8.shape[1]
    N = 2 * H

    def body(px_ref, py_ref, pz_ref, st_ref, out_ref,
             dist_scr, ax_scr, ay_scr, az_scr):
        px, py, pz = px_ref[...], py_ref[...], pz_ref[...]
        row = lax.broadcasted_iota(jnp.int32, (8, H), 0)
        io8 = lax.broadcasted_iota(jnp.int32, (8, H), 1) + \
            jnp.where(row >= B, H, 0)
        io_m = lax.broadcasted_iota(jnp.int32, (B, _M), 1)
        rowcol = lax.broadcasted_iota(jnp.int32, (8, 1), 0)

        st = st_ref[...]

        def step(i, carry):
            cxb, cyb, czb = carry
            ax_scr[...] = jnp.where(io_m == i, cxb, ax_scr[...])
            ay_scr[...] = jnp.where(io_m == i, cyb, ay_scr[...])
            az_scr[...] = jnp.where(io_m == i, czb, az_scr[...])
            cx8 = jnp.concatenate([cxb, cxb], axis=0)
            cy8 = jnp.concatenate([cyb, cyb], axis=0)
            cz8 = jnp.concatenate([czb, czb], axis=0)
            d = (px - cx8) ** 2 + (py - cy8) ** 2 + (pz - cz8) ** 2
            dist = jnp.minimum(dist_scr[...], d)
            dist_scr[...] = dist
            m8 = jnp.max(dist, axis=1, keepdims=True)
            a8 = jnp.argmax(dist, axis=1).astype(jnp.int32)[:, None]
            g8 = a8 + jnp.where(rowcol >= B, H, 0)
            nb = jnp.where(m8[B:] > m8[0:B], g8[B:], g8[0:B])
            n8b = jnp.concatenate([nb, nb], axis=0)
            mask = io8 == n8b
            sx = jnp.sum(jnp.where(mask, px, 0.0), axis=1, keepdims=True)
            sy = jnp.sum(jnp.where(mask, py, 0.0), axis=1, keepdims=True)
            sz = jnp.sum(jnp.where(mask, pz, 0.0), axis=1, keepdims=True)
            cxb = sx[0:B] + sx[B:]
            cyb = sy[0:B] + sy[B:]
            czb = sz[0:B] + sz[B:]
            return cxb, cyb, czb

        dist_scr[...] = jnp.full((8, H), jnp.inf, jnp.float32)
        lax.fori_loop(
            0, _M, step, (st[:, 0:1], st[:, 1:2], st[:, 2:3]))
        out_ref[...] = jnp.concatenate(
            [ax_scr[...][:, None, :], ay_scr[...][:, None, :],
             az_scr[...][:, None, :]], axis=1)

    return pl.pallas_call(
        body,
        out_shape=jax.ShapeDtypeStruct((B, 3, _M), jnp.float32),
        scratch_shapes=[pltpu.VMEM((8, H), jnp.float32),
                        pltpu.VMEM((B, _M), jnp.float32),
                        pltpu.VMEM((B, _M), jnp.float32),
                        pltpu.VMEM((B, _M), jnp.float32)],
    )(px8, py8, pz8, start_xyz)


def _ballq(xyzT, new_xyz):
    B, _, N = xyzT.shape
    CB = 256
    NB = _M // CB

    def body(xyzT_ref, q_ref, out_ref, D_scr):
        b = pl.program_id(0)
        p = xyzT_ref[0]
        q = q_ref[0]
        dx = q[:, 0:1] - p[0:1, :]
        dy = q[:, 1:2] - p[1:2, :]
        dz = q[:, 2:3] - p[2:3, :]
        D = dx * dx + dy * dy + dz * dz
        iota = lax.broadcasted_iota(jnp.int32, (CB, N), 1)
        cnt = jnp.sum(jnp.where(D <= _RADIUS2, 1, 0).astype(jnp.int32),
                      axis=1, keepdims=True)
        l32 = lax.broadcasted_iota(jnp.int32, (CB, _S), 1)

        D_scr[...] = D

        def step(i, acc):
            Dm = D_scr[...]
            am = jnp.argmin(Dm, axis=1).astype(jnp.int32)[:, None]
            acc = acc + jnp.where(l32 == i, am, 0)
            D_scr[...] = jnp.where(iota == am, jnp.inf, Dm)
            return acc

        idxs = lax.fori_loop(
            0, _S, step, jnp.zeros((CB, _S), jnp.int32))
        first = idxs[:, 0:1]
        sel = jnp.where(l32 >= cnt, first, idxs)
        out_ref[0] = sel + b * N

    return pl.pallas_call(
        body,
        grid=(B, NB),
        in_specs=[
            pl.BlockSpec((1, 3, N), lambda b, i: (b, 0, 0)),
            pl.BlockSpec((1, CB, 3), lambda b, i: (b, i, 0)),
        ],
        out_specs=pl.BlockSpec((1, CB, _S), lambda b, i: (b, i, 0)),
        out_shape=jax.ShapeDtypeStruct((B, _M, _S), jnp.int32),
        scratch_shapes=[pltpu.VMEM((CB, N), jnp.float32)],
    )(xyzT, new_xyz)


def _sc_gather(table, idx):
    V, Dw = table.shape
    R = idx.shape[0]
    NW = 32
    per_w = R // NW
    CH = 512
    NCH = per_w // CH
    mesh = plsc.VectorSubcoreMesh(core_axis_name="c", subcore_axis_name="s")

    @functools.partial(
        pl.kernel,
        mesh=mesh,
        out_type=jax.ShapeDtypeStruct((R, Dw), jnp.float32),
        scratch_types=[
            pltpu.VMEM((CH,), jnp.int32),
            pltpu.VMEM((CH, Dw), jnp.float32),
            pltpu.SemaphoreType.DMA,
        ],
    )
    def gk(table_hbm, idx_hbm, out_hbm, idx_v, rows_v, sem):
        wid = lax.axis_index("s") * 2 + lax.axis_index("c")
        base = wid * per_w

        def chunk(c, carry):
            off = pl.multiple_of(base + c * CH, CH)
            pltpu.sync_copy(idx_hbm.at[pl.ds(off, CH)], idx_v)
            pltpu.async_copy(table_hbm.at[idx_v], rows_v, sem).wait()
            pltpu.sync_copy(rows_v, out_hbm.at[pl.ds(off, CH)])
            return carry

        lax.fori_loop(0, NCH, chunk, 0)

    return gk(table, idx)


def _mm1(x80, ctrp, w1p, w1a8, b1):
    R, K = x80.shape
    RB = 8192
    NB = R // RB
    G = RB // _S

    def body(x_ref, c_ref, w_ref, wa_ref, b_ref, y_ref, s_ref, q_ref):
        i = pl.program_id(0)
        y = jnp.dot(x_ref[...], w_ref[...],
                    preferred_element_type=jnp.float32)
        cy = jnp.dot(c_ref[...], wa_ref[...],
                     preferred_element_type=jnp.float32)
        y = (y.reshape(G, _S, 128) - cy[:, None, :]).reshape(RB, 128)
        y = y + b_ref[...]
        y_ref[...] = y
        ps = jnp.sum(y.reshape(RB // 8, 8, 128), axis=0)
        pq = jnp.sum((y * y).reshape(RB // 8, 8, 128), axis=0)

        @pl.when(i == 0)
        def _():
            s_ref[...] = ps
            q_ref[...] = pq

        @pl.when(i > 0)
        def _():
            s_ref[...] += ps
            q_ref[...] += pq

    return pl.pallas_call(
        body,
        grid=(NB,),
        in_specs=[
            pl.BlockSpec((RB, K), lambda i: (i, 0)),
            pl.BlockSpec((G, 8), lambda i: (i, 0)),
            pl.BlockSpec((K, 128), lambda i: (0, 0)),
            pl.BlockSpec((8, 128), lambda i: (0, 0)),
            pl.BlockSpec((1, 128), lambda i: (0, 0)),
        ],
        out_specs=[
            pl.BlockSpec((RB, 128), lambda i: (i, 0)),
            pl.BlockSpec((8, 128), lambda i: (0, 0)),
            pl.BlockSpec((8, 128), lambda i: (0, 0)),
        ],
        out_shape=[
            jax.ShapeDtypeStruct((R, 128), jnp.float32),
            jax.ShapeDtypeStruct((8, 128), jnp.float32),
            jax.ShapeDtypeStruct((8, 128), jnp.float32),
        ],
    )(x80, ctrp, w1p, w1a8, b1)


def _mmk(y_in, a, bc, wt, bias):
    R, K = y_in.shape
    Nc = wt.shape[1]
    RB = 8192
    NB = R // RB

    def body(x_ref, a_ref, c_ref, w_ref, b_ref, y_ref, s_ref, q_ref):
        i = pl.program_id(0)
        z = jnp.maximum(x_ref[...] * a_ref[...] + c_ref[...], 0.0)
        y = jnp.dot(z, w_ref[...], preferred_element_type=jnp.float32)
        y = y + b_ref[...]
        y_ref[...] = y
        ps = jnp.sum(y.reshape(RB // 8, 8, Nc), axis=0)
        pq = jnp.sum((y * y).reshape(RB // 8, 8, Nc), axis=0)

        @pl.when(i == 0)
        def _():
            s_ref[...] = ps
            q_ref[...] = pq

        @pl.when(i > 0)
        def _():
            s_ref[...] += ps
            q_ref[...] += pq

    return pl.pallas_call(
        body,
        grid=(NB,),
        in_specs=[
            pl.BlockSpec((RB, K), lambda i: (i, 0)),
            pl.BlockSpec((1, K), lambda i: (0, 0)),
            pl.BlockSpec((1, K), lambda i: (0, 0)),
            pl.BlockSpec((K, Nc), lambda i: (0, 0)),
            pl.BlockSpec((1, Nc), lambda i: (0, 0)),
        ],
        out_specs=[
            pl.BlockSpec((RB, Nc), lambda i: (i, 0)),
            pl.BlockSpec((8, Nc), lambda i: (0, 0)),
            pl.BlockSpec((8, Nc), lambda i: (0, 0)),
        ],
        out_shape=[
            jax.ShapeDtypeStruct((R, Nc), jnp.float32),
            jax.ShapeDtypeStruct((8, Nc), jnp.float32),
            jax.ShapeDtypeStruct((8, Nc), jnp.float32),
        ],
    )(y_in, a, bc, wt, bias)


def _final(y3, a, bc):
    R, Nc = y3.shape
    RB = 8192
    NB = R // RB
    G = RB // _S

    def body(x_ref, a_ref, c_ref, out_ref):
        z = jnp.maximum(x_ref[...] * a_ref[...] + c_ref[...], 0.0)
        out_ref[...] = jnp.max(z.reshape(G, _S, Nc), axis=1)

    return pl.pallas_call(
        body,
        grid=(NB,),
        in_specs=[
            pl.BlockSpec((RB, Nc), lambda i: (i, 0)),
            pl.BlockSpec((1, Nc), lambda i: (0, 0)),
            pl.BlockSpec((1, Nc), lambda i: (0, 0)),
        ],
        out_specs=pl.BlockSpec((G, Nc), lambda i: (i, 0)),
        out_shape=jax.ShapeDtypeStruct((R // _S, Nc), jnp.float32),
    )(y3, a, bc)


def kernel(xyz, features, W1, b1, g1, beta1, W2, b2, g2, beta2,
           W3, b3, g3, beta3):
    B, N, _ = xyz.shape
    C = features.shape[-1]

    xyzT = jnp.transpose(xyz, (0, 2, 1))
    idx0 = jax.random.randint(jax.random.key(7), (B,), 0, N)
    start_xyz = xyz[jnp.arange(B), idx0]

    H = N // 2
    planes8 = jnp.transpose(xyzT.reshape(B, 3, 2, H),
                            (1, 2, 0, 3)).reshape(3, 2 * B, H)
    oxyzT = _fps(planes8[0], planes8[1], planes8[2], start_xyz)
    new_xyz = jnp.transpose(oxyzT, (0, 2, 1))

    ball_idx = _ballq(xyzT, new_xyz)

    TW = 128
    table = jnp.concatenate(
        [xyz, jnp.zeros((B, N, 13), jnp.float32), features,
         jnp.zeros((B, N, TW - 16 - C), jnp.float32)],
        axis=-1).reshape(B * N, TW)
    gathered = _sc_gather(table, ball_idx.reshape(-1))

    ctrp = jnp.pad(new_xyz.reshape(B * _M, 3), ((0, 0), (0, 5)))
    w1p = jnp.zeros((TW, 128), jnp.float32)
    w1p = w1p.at[0:3].set(W1[:, 0:3].T).at[16:16 + C].set(W1[:, 3:3 + C].T)
    w1a8 = jnp.zeros((8, 128), jnp.float32).at[0:3].set(W1[:, 0:3].T)

    cnt = float(B * _M * _S)

    def fold(s, q, g, beta):
        mean = jnp.sum(s, axis=0) / cnt
        var = jnp.sum(q, axis=0) / cnt - mean * mean
        a = g / jnp.sqrt(var + _EPS)
        return (a.reshape(1, -1), (beta - mean * a).reshape(1, -1))

    y1, s1, q1 = _mm1(gathered, ctrp, w1p, w1a8, b1.reshape(1, -1))
    a1, c1 = fold(s1, q1, g1, beta1)
    y2, s2, q2 = _mmk(y1, a1, c1, W2.T, b2.reshape(1, -1))
    a2, c2 = fold(s2, q2, g2, beta2)
    y3, s3, q3 = _mmk(y2, a2, c2, W3.T, b3.reshape(1, -1))
    a3, c3 = fold(s3, q3, g3, beta3)
    feat_out = _final(y3, a3, c3).reshape(B, _M, W3.shape[0])

    return new_xyz, feat_out


# speedup vs baseline: 1.1585x; 1.0030x over previous
import functools

import jax
import jax.numpy as jnp
from jax import lax
from jax.experimental import pallas as pl
from jax.experimental.pallas import tpu as pltpu
from jax.experimental.pallas import tpu_sc as plsc

_M = 1024
_S = 32
_RADIUS2 = 0.2 ** 2
_EPS = 1e-5


def _fps(px8, py8, pz8, start_xyz):
    B = 4
    H = px8.shape[1]
    N = 2 * H

    def body(px_ref, py_ref, pz_ref, st_ref, out_ref,
             dist_scr, ax_scr, ay_scr, az_scr):
        px, py, pz = px_ref[...], py_ref[...], pz_ref[...]
        row = lax.broadcasted_iota(jnp.int32, (8, H), 0)
        io8 = lax.broadcasted_iota(jnp.int32, (8, H), 1) + \
            jnp.where(row >= B, H, 0)
        io_m = lax.broadcasted_iota(jnp.int32, (B, _M), 1)
        rowcol = lax.broadcasted_iota(jnp.int32, (8, 1), 0)

        st = st_ref[...]

        def step(i, carry):
            cxb, cyb, czb = carry
            ax_scr[...] = jnp.where(io_m == i, cxb, ax_scr[...])
            ay_scr[...] = jnp.where(io_m == i, cyb, ay_scr[...])
            az_scr[...] = jnp.where(io_m == i, czb, az_scr[...])
            cx8 = jnp.concatenate([cxb, cxb], axis=0)
            cy8 = jnp.concatenate([cyb, cyb], axis=0)
            cz8 = jnp.concatenate([czb, czb], axis=0)
            d = (px - cx8) ** 2 + (py - cy8) ** 2 + (pz - cz8) ** 2
            dist = jnp.minimum(dist_scr[...], d)
            dist_scr[...] = dist
            m8 = jnp.max(dist, axis=1, keepdims=True)
            a8 = jnp.argmax(dist, axis=1).astype(jnp.int32)[:, None]
            g8 = a8 + jnp.where(rowcol >= B, H, 0)
            nb = jnp.where(m8[B:] > m8[0:B], g8[B:], g8[0:B])
            n8b = jnp.concatenate([nb, nb], axis=0)
            mask = io8 == n8b
            sx = jnp.sum(jnp.where(mask, px, 0.0), axis=1, keepdims=True)
            sy = jnp.sum(jnp.where(mask, py, 0.0), axis=1, keepdims=True)
            sz = jnp.sum(jnp.where(mask, pz, 0.0), axis=1, keepdims=True)
            cxb = sx[0:B] + sx[B:]
            cyb = sy[0:B] + sy[B:]
            czb = sz[0:B] + sz[B:]
            return cxb, cyb, czb

        dist_scr[...] = jnp.full((8, H), jnp.inf, jnp.float32)
        lax.fori_loop(
            0, _M, step, (st[:, 0:1], st[:, 1:2], st[:, 2:3]))
        out_ref[...] = jnp.concatenate(
            [ax_scr[...][:, None, :], ay_scr[...][:, None, :],
             az_scr[...][:, None, :]], axis=1)

    return pl.pallas_call(
        body,
        out_shape=jax.ShapeDtypeStruct((B, 3, _M), jnp.float32),
        scratch_shapes=[pltpu.VMEM((8, H), jnp.float32),
                        pltpu.VMEM((B, _M), jnp.float32),
                        pltpu.VMEM((B, _M), jnp.float32),
                        pltpu.VMEM((B, _M), jnp.float32)],
    )(px8, py8, pz8, start_xyz)


def _ballq(xyzT, new_xyz):
    B, _, N = xyzT.shape
    CB = 256
    NB = _M // CB

    def body(xyzT_ref, q_ref, out_ref, D_scr):
        b = pl.program_id(0)
        p = xyzT_ref[0]
        q = q_ref[0]
        dx = q[:, 0:1] - p[0:1, :]
        dy = q[:, 1:2] - p[1:2, :]
        dz = q[:, 2:3] - p[2:3, :]
        D = dx * dx + dy * dy + dz * dz
        iota = lax.broadcasted_iota(jnp.int32, (CB, N), 1)
        cnt = jnp.sum(jnp.where(D <= _RADIUS2, 1, 0).astype(jnp.int32),
                      axis=1, keepdims=True)
        l32 = lax.broadcasted_iota(jnp.int32, (CB, _S), 1)

        D_scr[...] = D

        def step(i, acc):
            Dm = D_scr[...]
            am = jnp.argmin(Dm, axis=1).astype(jnp.int32)[:, None]
            acc = acc + jnp.where(l32 == i, am, 0)
            D_scr[...] = jnp.where(iota == am, jnp.inf, Dm)
            return acc

        idxs = lax.fori_loop(
            0, _S, step, jnp.zeros((CB, _S), jnp.int32))
        first = idxs[:, 0:1]
        sel = jnp.where(l32 >= cnt, first, idxs)
        out_ref[0] = sel + b * N

    return pl.pallas_call(
        body,
        grid=(B, NB),
        in_specs=[
            pl.BlockSpec((1, 3, N), lambda b, i: (b, 0, 0)),
            pl.BlockSpec((1, CB, 3), lambda b, i: (b, i, 0)),
        ],
        out_specs=pl.BlockSpec((1, CB, _S), lambda b, i: (b, i, 0)),
        out_shape=jax.ShapeDtypeStruct((B, _M, _S), jnp.int32),
        scratch_shapes=[pltpu.VMEM((CB, N), jnp.float32)],
    )(xyzT, new_xyz)


def _sc_gather(table, idx):
    V, Dw = table.shape
    R = idx.shape[0]
    NW = 32
    per_w = R // NW
    CH = 512
    NCH = per_w // CH
    mesh = plsc.VectorSubcoreMesh(core_axis_name="c", subcore_axis_name="s")

    @functools.partial(
        pl.kernel,
        mesh=mesh,
        out_type=jax.ShapeDtypeStruct((R, Dw), jnp.float32),
        scratch_types=[
            pltpu.VMEM((CH,), jnp.int32),
            pltpu.VMEM((CH, Dw), jnp.float32),
            pltpu.SemaphoreType.DMA,
        ],
    )
    def gk(table_hbm, idx_hbm, out_hbm, idx_v, rows_v, sem):
        wid = lax.axis_index("s") * 2 + lax.axis_index("c")
        base = wid * per_w

        def chunk(c, carry):
            off = pl.multiple_of(base + c * CH, CH)
            pltpu.sync_copy(idx_hbm.at[pl.ds(off, CH)], idx_v)
            pltpu.async_copy(table_hbm.at[idx_v], rows_v, sem).wait()
            pltpu.sync_copy(rows_v, out_hbm.at[pl.ds(off, CH)])
            return carry

        lax.fori_loop(0, NCH, chunk, 0)

    return gk(table, idx)


def _mm1(x80, ctrp, w1p, w1a8, b1):
    R, K = x80.shape
    RB = 8192
    NB = R // RB
    G = RB // _S

    def body(x_ref, c_ref, w_ref, wa_ref, b_ref, y_ref, s_ref, q_ref):
        i = pl.program_id(0)
        y = jnp.dot(x_ref[...], w_ref[...],
                    preferred_element_type=jnp.float32)
        cy = jnp.dot(c_ref[...], wa_ref[...],
                     preferred_element_type=jnp.float32)
        y = (y.reshape(G, _S, 128) - cy[:, None, :]).reshape(RB, 128)
        y = y + b_ref[...]
        y_ref[...] = y
        ps = jnp.sum(y.reshape(RB // 8, 8, 128), axis=0)
        pq = jnp.sum((y * y).reshape(RB // 8, 8, 128), axis=0)

        @pl.when(i == 0)
        def _():
            s_ref[...] = ps
            q_ref[...] = pq

        @pl.when(i > 0)
        def _():
            s_ref[...] += ps
            q_ref[...] += pq

    return pl.pallas_call(
        body,
        grid=(NB,),
        in_specs=[
            pl.BlockSpec((RB, K), lambda i: (i, 0)),
            pl.BlockSpec((G, 8), lambda i: (i, 0)),
            pl.BlockSpec((K, 128), lambda i: (0, 0)),
            pl.BlockSpec((8, 128), lambda i: (0, 0)),
            pl.BlockSpec((1, 128), lambda i: (0, 0)),
        ],
        out_specs=[
            pl.BlockSpec((RB, 128), lambda i: (i, 0)),
            pl.BlockSpec((8, 128), lambda i: (0, 0)),
            pl.BlockSpec((8, 128), lambda i: (0, 0)),
        ],
        out_shape=[
            jax.ShapeDtypeStruct((R, 128), jnp.float32),
            jax.ShapeDtypeStruct((8, 128), jnp.float32),
            jax.ShapeDtypeStruct((8, 128), jnp.float32),
        ],
    )(x80, ctrp, w1p, w1a8, b1)


def _mmk(y_in, s_in, q_in, g, beta, wt, bias):
    R, K = y_in.shape
    Nc = wt.shape[1]
    RB = 8192
    NB = R // RB

    def body(x_ref, sp_ref, qp_ref, g_ref, be_ref, w_ref, b_ref,
             y_ref, s_ref, q_ref):
        i = pl.program_id(0)
        mean = jnp.sum(sp_ref[...], axis=0, keepdims=True) / R
        var = jnp.sum(qp_ref[...], axis=0, keepdims=True) / R - mean * mean
        a = g_ref[...] / jnp.sqrt(var + _EPS)
        c = be_ref[...] - mean * a
        z = jnp.maximum(x_ref[...] * a + c, 0.0)
        y = jnp.dot(z, w_ref[...], preferred_element_type=jnp.float32)
        y = y + b_ref[...]
        y_ref[...] = y
        ps = jnp.sum(y.reshape(RB // 8, 8, Nc), axis=0)
        pq = jnp.sum((y * y).reshape(RB // 8, 8, Nc), axis=0)

        @pl.when(i == 0)
        def _():
            s_ref[...] = ps
            q_ref[...] = pq

        @pl.when(i > 0)
        def _():
            s_ref[...] += ps
            q_ref[...] += pq

    return pl.pallas_call(
        body,
        grid=(NB,),
        in_specs=[
            pl.BlockSpec((RB, K), lambda i: (i, 0)),
            pl.BlockSpec((8, K), lambda i: (0, 0)),
            pl.BlockSpec((8, K), lambda i: (0, 0)),
            pl.BlockSpec((1, K), lambda i: (0, 0)),
            pl.BlockSpec((1, K), lambda i: (0, 0)),
            pl.BlockSpec((K, Nc), lambda i: (0, 0)),
            pl.BlockSpec((1, Nc), lambda i: (0, 0)),
        ],
        out_specs=[
            pl.BlockSpec((RB, Nc), lambda i: (i, 0)),
            pl.BlockSpec((8, Nc), lambda i: (0, 0)),
            pl.BlockSpec((8, Nc), lambda i: (0, 0)),
        ],
        out_shape=[
            jax.ShapeDtypeStruct((R, Nc), jnp.float32),
            jax.ShapeDtypeStruct((8, Nc), jnp.float32),
            jax.ShapeDtypeStruct((8, Nc), jnp.float32),
        ],
    )(y_in, s_in, q_in, g, beta, wt, bias)


def _final(y3, s_in, q_in, g, beta):
    R, Nc = y3.shape
    RB = 8192
    NB = R // RB
    G = RB // _S

    def body(x_ref, sp_ref, qp_ref, g_ref, be_ref, out_ref):
        mean = jnp.sum(sp_ref[...], axis=0, keepdims=True) / R
        var = jnp.sum(qp_ref[...], axis=0, keepdims=True) / R - mean * mean
        a = g_ref[...] / jnp.sqrt(var + _EPS)
        c = be_ref[...] - mean * a
        z = jnp.maximum(x_ref[...] * a + c, 0.0)
        out_ref[...] = jnp.max(z.reshape(G, _S, Nc), axis=1)

    return pl.pallas_call(
        body,
        grid=(NB,),
        in_specs=[
            pl.BlockSpec((RB, Nc), lambda i: (i, 0)),
            pl.BlockSpec((8, Nc), lambda i: (0, 0)),
            pl.BlockSpec((8, Nc), lambda i: (0, 0)),
            pl.BlockSpec((1, Nc), lambda i: (0, 0)),
            pl.BlockSpec((1, Nc), lambda i: (0, 0)),
        ],
        out_specs=pl.BlockSpec((G, Nc), lambda i: (i, 0)),
        out_shape=jax.ShapeDtypeStruct((R // _S, Nc), jnp.float32),
    )(y3, s_in, q_in, g, beta)


def kernel(xyz, features, W1, b1, g1, beta1, W2, b2, g2, beta2,
           W3, b3, g3, beta3):
    B, N, _ = xyz.shape
    C = features.shape[-1]

    xyzT = jnp.transpose(xyz, (0, 2, 1))
    idx0 = jax.random.randint(jax.random.key(7), (B,), 0, N)
    start_xyz = xyz[jnp.arange(B), idx0]

    H = N // 2
    planes8 = jnp.transpose(xyzT.reshape(B, 3, 2, H),
                            (1, 2, 0, 3)).reshape(3, 2 * B, H)
    oxyzT = _fps(planes8[0], planes8[1], planes8[2], start_xyz)
    new_xyz = jnp.transpose(oxyzT, (0, 2, 1))

    ball_idx = _ballq(xyzT, new_xyz)

    TW = 128
    table = jnp.concatenate(
        [xyz, jnp.zeros((B, N, 13), jnp.float32), features,
         jnp.zeros((B, N, TW - 16 - C), jnp.float32)],
        axis=-1).reshape(B * N, TW)
    gathered = _sc_gather(table, ball_idx.reshape(-1))

    ctrp = jnp.pad(new_xyz.reshape(B * _M, 3), ((0, 0), (0, 5)))
    w1p = jnp.zeros((TW, 128), jnp.float32)
    w1p = w1p.at[0:3].set(W1[:, 0:3].T).at[16:16 + C].set(W1[:, 3:3 + C].T)
    w1a8 = jnp.zeros((8, 128), jnp.float32).at[0:3].set(W1[:, 0:3].T)

    y1, s1, q1 = _mm1(gathered, ctrp, w1p, w1a8, b1.reshape(1, -1))
    y2, s2, q2 = _mmk(y1, s1, q1, g1.reshape(1, -1), beta1.reshape(1, -1),
                      W2.T, b2.reshape(1, -1))
    y3, s3, q3 = _mmk(y2, s2, q2, g2.reshape(1, -1), beta2.reshape(1, -1),
                      W3.T, b3.reshape(1, -1))
    feat_out = _final(y3, s3, q3, g3.reshape(1, -1),
                      beta3.reshape(1, -1)).reshape(B, _M, W3.shape[0])

    return new_xyz, feat_out
